# baseline (speedup 1.0000x reference)
import functools

import jax
import jax.numpy as jnp
from jax import lax
from jax.experimental import pallas as pl
from jax.experimental.pallas import tpu as pltpu
from jax.experimental.pallas import tpu_sc as plsc

B = 16384
F = 64
N_ROWS = 1000000

_info = plsc.get_sparse_core_info()
NC = _info.num_cores
NS = _info.num_subcores
L = _info.num_lanes
NW = NC * NS
BPW = B // NW
CH = 128
NCH = BPW // CH

_mesh = plsc.VectorSubcoreMesh(core_axis_name="c", subcore_axis_name="s")


@functools.partial(
    pl.kernel,
    mesh=_mesh,
    compiler_params=pltpu.CompilerParams(
        needs_layout_passes=False, use_tc_tiling_on_sc=False),
    out_type=jax.ShapeDtypeStruct((B,), jnp.float32),
    scratch_types=[
        pltpu.VMEM((BPW,), jnp.int32),
        pltpu.VMEM((BPW,), jnp.int32),
        pltpu.VMEM((NCH, CH), jnp.int32),
        pltpu.VMEM((NCH, CH), jnp.int32),
        pltpu.VMEM((NCH, CH), jnp.float32),
        pltpu.VMEM((NCH, CH), jnp.float32),
        pltpu.VMEM((BPW,), jnp.float32),
        pltpu.SemaphoreType.DMA,
    ],
)
def _mf_sc(user_hbm, item_hbm, ufT_hbm, ifT_hbm, out_hbm,
           ubase, ibase, pu, pi, ue, ie, outv, sem):
    wid = lax.axis_index("s") * NC + lax.axis_index("c")

    pltpu.sync_copy(user_hbm.at[wid], ubase)
    pltpu.sync_copy(item_hbm.at[wid], ibase)

    def zero_body(j, _):
        outv[pl.ds(j * L, L)] = jnp.zeros((L,), jnp.float32)
        return 0

    lax.fori_loop(0, BPW // L, zero_body, 0)

    ufT0 = ufT_hbm.at[0]
    ifT0 = ifT_hbm.at[0]

    def c_body(c, _):
        coff = c * N_ROWS

        def idx_body(k, _):
            def idx_inner(p, _):
                sl = pl.ds(p * L, L)
                bsl = pl.ds(k * CH + p * L, L)
                pu[k, sl] = ubase[bsl] + coff
                pi[k, sl] = ibase[bsl] + coff
                return 0
            lax.fori_loop(0, CH // L, idx_inner, 0)
            return 0

        lax.fori_loop(0, NCH, idx_body, 0)

        copies = []
        for k in range(NCH):
            copies.append(pltpu.async_copy(ufT0.at[pu.at[k]], ue.at[k], sem))
            copies.append(pltpu.async_copy(ifT0.at[pi.at[k]], ie.at[k], sem))
        for cp in copies:
            cp.wait()

        def acc_body(k, _):
            def acc_inner(p, _):
                sl = pl.ds(p * L, L)
                osl = pl.ds(k * CH + p * L, L)
                outv[osl] = outv[osl] + ue[k, sl] * ie[k, sl]
                return 0
            lax.fori_loop(0, CH // L, acc_inner, 0)
            return 0

        lax.fori_loop(0, NCH, acc_body, 0)
        return 0

    lax.fori_loop(0, F, c_body, 0)

    pltpu.sync_copy(outv, out_hbm.at[pl.ds(wid * BPW, BPW)])


def kernel(user, item, user_factors, item_factors):
    user_r = user.astype(jnp.int32).reshape(NW, BPW)
    item_r = item.astype(jnp.int32).reshape(NW, BPW)
    return _mf_sc(user_r, item_r, user_factors.T, item_factors.T)
